# Initial kernel scaffold; baseline (speedup 1.0000x reference)
import jax
import jax.numpy as jnp
from jax.experimental import pallas as pl


def kernel(x, edge_index, W1, b1, W2, b2, W3, b3, W4, b4, Wout, bout):
    raise NotImplementedError("write your pallas kernel here")


# speedup vs baseline: 6.4079x; 6.4079x over previous
import functools

import jax
import jax.numpy as jnp
from jax import lax
from jax.experimental import pallas as pl
from jax.experimental.pallas import tpu as pltpu
from jax.experimental.pallas import tpu_sc as plsc

N = 10000
D = 128
OUT_C = 12
N_OUT = 2000

N_PAD = 10240
E = 320000
CH = 128
TILES = 32
E_PAD = 327680
EPT = E_PAD // TILES
NCH = EPT // CH
TPC = 16
RPT = N_PAD // TPC

_sc_cache = {}


def _build_sc_kernels():
    if _sc_cache:
        return _sc_cache["deg"], _sc_cache["scat"]
    mesh = plsc.VectorSubcoreMesh(core_axis_name="c", subcore_axis_name="s")

    @functools.partial(
        pl.kernel,
        out_type=jax.ShapeDtypeStruct((2, N_PAD, D), jnp.float32),
        mesh=mesh,
        scratch_types=[
            pltpu.VMEM((NCH, CH), jnp.int32),
            pltpu.VMEM((CH, D), jnp.float32),
            pltpu.VMEM_SHARED((N_PAD, D), jnp.float32),
        ],
    )
    def sc_degree(dst_hbm, ones_hbm, zeros_hbm, out_hbm, dst_v, ones_v, acc):
        c = lax.axis_index("c")
        s = lax.axis_index("s")
        tile = c * TPC + s
        pltpu.sync_copy(dst_hbm.at[pl.ds(tile * NCH, NCH)], dst_v)
        pltpu.sync_copy(ones_hbm, ones_v)
        pltpu.sync_copy(zeros_hbm, acc.at[pl.ds(s * RPT, RPT)])
        plsc.subcore_barrier()

        def body(j, carry):
            pltpu.sync_copy(ones_v, acc.at[dst_v.at[j]], add=True)
            return carry

        lax.fori_loop(0, NCH, body, 0)
        plsc.subcore_barrier()
        pltpu.sync_copy(acc.at[pl.ds(s * RPT, RPT)],
                        out_hbm.at[c].at[pl.ds(s * RPT, RPT)])

    @functools.partial(
        pl.kernel,
        out_type=jax.ShapeDtypeStruct((2, N_PAD, D), jnp.float32),
        mesh=mesh,
        scratch_types=[
            pltpu.VMEM((NCH, CH), jnp.int32),
            pltpu.VMEM((NCH, CH), jnp.int32),
            pltpu.VMEM((CH, D), jnp.float32),
            pltpu.VMEM_SHARED((N_PAD, D), jnp.float32),
            pltpu.SemaphoreType.DMA,
        ],
    )
    def sc_scatter(g_hbm, src_hbm, dst_hbm, zeros_hbm, out_hbm,
                   src_v, dst_v, rows_v, acc, sem):
        c = lax.axis_index("c")
        s = lax.axis_index("s")
        tile = c * TPC + s
        pltpu.sync_copy(src_hbm.at[pl.ds(tile * NCH, NCH)], src_v)
        pltpu.sync_copy(dst_hbm.at[pl.ds(tile * NCH, NCH)], dst_v)
        pltpu.sync_copy(zeros_hbm, acc.at[pl.ds(s * RPT, RPT)])
        plsc.subcore_barrier()

        def body(j, carry):
            pltpu.async_copy(g_hbm.at[src_v.at[j]], rows_v, sem).wait()
            pltpu.sync_copy(rows_v, acc.at[dst_v.at[j]], add=True)
            return carry

        lax.fori_loop(0, NCH, body, 0)
        plsc.subcore_barrier()
        pltpu.sync_copy(acc.at[pl.ds(s * RPT, RPT)],
                        out_hbm.at[c].at[pl.ds(s * RPT, RPT)])

    _sc_cache["deg"] = sc_degree
    _sc_cache["scat"] = sc_scatter
    return sc_degree, sc_scatter


def _sc_degree(dst2, ones_big, zeros_big):
    return _build_sc_kernels()[0](dst2, ones_big, zeros_big)


def _sc_scatter(g, src2, dst2, zeros_big):
    return _build_sc_kernels()[1](g, src2, dst2, zeros_big)


_BLK = 1280
_GRID = N_PAD // _BLK


def _tc_layer1_body(x_ref, w_ref, deg_ref, g_ref, dis_ref):
    i = pl.program_id(0)
    rows = i * _BLK + lax.broadcasted_iota(jnp.int32, (_BLK, 1), 0)
    cnt = deg_ref[0, :, 0:1] + deg_ref[1, :, 0:1] + 1.0
    dis = lax.rsqrt(cnt) * (rows < N).astype(jnp.float32)
    dis_ref[...] = jnp.broadcast_to(dis, (_BLK, 16))
    h = jnp.dot(x_ref[...], w_ref[...], preferred_element_type=jnp.float32)
    g_ref[...] = h * dis


def _tc_layer1(x_pad, w1, deg2):
    return pl.pallas_call(
        _tc_layer1_body,
        grid=(_GRID,),
        in_specs=[
            pl.BlockSpec((_BLK, D), lambda i: (i, 0)),
            pl.BlockSpec((D, D), lambda i: (0, 0)),
            pl.BlockSpec((2, _BLK, D), lambda i: (0, i, 0)),
        ],
        out_specs=[
            pl.BlockSpec((_BLK, D), lambda i: (i, 0)),
            pl.BlockSpec((_BLK, 16), lambda i: (i, 0)),
        ],
        out_shape=[
            jax.ShapeDtypeStruct((N_PAD, D), jnp.float32),
            jax.ShapeDtypeStruct((N_PAD, 16), jnp.float32),
        ],
    )(x_pad, w1, deg2)


def _tc_mid_body(s_ref, g_ref, dis_ref, b_ref, w_ref, out_ref):
    d = dis_ref[:, 0:1]
    xk = (s_ref[0] + s_ref[1] + g_ref[...]) * d + b_ref[0]
    xk = jnp.maximum(xk, 0.0)
    h = jnp.dot(xk, w_ref[...], preferred_element_type=jnp.float32)
    out_ref[...] = h * d


def _tc_mid(s2, g_prev, dis16, b_prev, w_next):
    return pl.pallas_call(
        _tc_mid_body,
        grid=(_GRID,),
        in_specs=[
            pl.BlockSpec((2, _BLK, D), lambda i: (0, i, 0)),
            pl.BlockSpec((_BLK, D), lambda i: (i, 0)),
            pl.BlockSpec((_BLK, 16), lambda i: (i, 0)),
            pl.BlockSpec((1, D), lambda i: (0, 0)),
            pl.BlockSpec((D, D), lambda i: (0, 0)),
        ],
        out_specs=pl.BlockSpec((_BLK, D), lambda i: (i, 0)),
        out_shape=jax.ShapeDtypeStruct((N_PAD, D), jnp.float32),
    )(s2, g_prev, dis16, b_prev, w_next)


def _tc_final_body(s_ref, g_ref, dis_ref, b_ref, wout_ref, bout_ref, out_ref):
    d = dis_ref[:, 0:1]
    h = (s_ref[0] + s_ref[1] + g_ref[...]) * d + b_ref[0]
    h = jnp.maximum(h, 0.0)
    out_ref[...] = (
        jnp.dot(h, wout_ref[...], preferred_element_type=jnp.float32)
        + bout_ref[0]
    )


def _tc_final(s2, g4, dis16, b4, wout_pad, bout_pad):
    return pl.pallas_call(
        _tc_final_body,
        grid=(1,),
        in_specs=[
            pl.BlockSpec((2, N_OUT, D), lambda i: (0, 4, 0)),
            pl.BlockSpec((N_OUT, D), lambda i: (4, 0)),
            pl.BlockSpec((N_OUT, 16), lambda i: (4, 0)),
            pl.BlockSpec((1, D), lambda i: (0, 0)),
            pl.BlockSpec((D, D), lambda i: (0, 0)),
            pl.BlockSpec((1, D), lambda i: (0, 0)),
        ],
        out_specs=pl.BlockSpec((N_OUT, D), lambda i: (0, 0)),
        out_shape=jax.ShapeDtypeStruct((N_OUT, D), jnp.float32),
    )(s2, g4, dis16, b4, wout_pad, bout_pad)


@jax.jit
def _run(x, edge_index, W1, b1, W2, b2, W3, b3, W4, b4, Wout, bout):
    src = edge_index[0].astype(jnp.int32)
    dst = edge_index[1].astype(jnp.int32)
    npad = E_PAD - E
    src_p = jnp.concatenate([src, jnp.full((npad,), N, jnp.int32)])
    dst_p = jnp.concatenate([dst, jnp.full((npad,), N_PAD - 1, jnp.int32)])
    src2 = src_p.reshape(E_PAD // CH, CH)
    dst2 = dst_p.reshape(E_PAD // CH, CH)

    x_pad = jnp.pad(x, ((0, N_PAD - N), (0, 0)))
    zeros_big = jnp.zeros((RPT, D), jnp.float32)
    ones_big = jnp.ones((CH, D), jnp.float32)

    deg2 = _sc_degree(dst2, ones_big, zeros_big)
    g, dis16 = _tc_layer1(x_pad, W1, deg2)

    b1r = b1.reshape(1, D)
    b2r = b2.reshape(1, D)
    b3r = b3.reshape(1, D)
    for w_next, b_prev in ((W2, b1r), (W3, b2r), (W4, b3r)):
        s2 = _sc_scatter(g, src2, dst2, zeros_big)
        g = _tc_mid(s2, g, dis16, b_prev, w_next)

    s2 = _sc_scatter(g, src2, dst2, zeros_big)
    wout_pad = jnp.pad(Wout, ((0, 0), (0, D - OUT_C)))
    bout_pad = jnp.pad(bout, (0, D - OUT_C)).reshape(1, D)
    outp = _tc_final(s2, g, dis16, b4.reshape(1, D), wout_pad, bout_pad)
    return outp[:, :OUT_C]


def kernel(x, edge_index, W1, b1, W2, b2, W3, b3, W4, b4, Wout, bout):
    return _run(x, edge_index, W1, b1, W2, b2, W3, b3, W4, b4, Wout, bout)
